# Initial kernel scaffold; baseline (speedup 1.0000x reference)
import jax
import jax.numpy as jnp
from jax.experimental import pallas as pl


def kernel(x, edge_index, batch, W1, b1, W2, b2, Wfc, bfc):
    raise NotImplementedError("write your pallas kernel here")


# speedup vs baseline: 18.0459x; 18.0459x over previous
import functools

import jax
import jax.numpy as jnp
from jax import lax
from jax.experimental import pallas as pl
from jax.experimental.pallas import tpu as pltpu
from jax.experimental.pallas import tpu_sc as plsc

N = 50000
E = 800000
D_IN = 128
D_H = 64
HALF = D_H // 2
N_GRAPHS = 64

NC = 2
NS = 16

K = 128
SB = 8
E_PAD = ((E + NC * NS * K - 1) // (NC * NS * K)) * (NC * NS * K)
NROWS_E = E_PAD // K
NPAD = 50176
TPW = NPAD // NS

BLK = 2000
NBLK = N // BLK

_mesh = plsc.VectorSubcoreMesh(core_axis_name="c", subcore_axis_name="s")


@functools.partial(
    pl.kernel,
    mesh=_mesh,
    out_type=jax.ShapeDtypeStruct((NC * NPAD,), jnp.float32),
    scratch_types=[
        pltpu.VMEM((4, K), jnp.int32),
        pltpu.VMEM((K,), jnp.float32),
        pltpu.VMEM((TPW,), jnp.float32),
        pltpu.VMEM_SHARED((NPAD,), jnp.float32),
    ],
)
def _deg_kernel(dst2d, out, idx_v, ones_v, zbuf, dacc):
    cc = lax.axis_index("c")
    ss = lax.axis_index("s")
    z16 = jnp.zeros((16,), jnp.float32)
    o16 = jnp.ones((16,), jnp.float32)

    def initz(i, _):
        zbuf[pl.ds(i * 16, 16)] = z16
        return 0

    lax.fori_loop(0, TPW // 16, initz, 0)
    for i in range(K // 16):
        ones_v[pl.ds(i * 16, 16)] = o16

    pltpu.sync_copy(zbuf, dacc.at[pl.ds(ss * TPW, TPW)])
    plsc.subcore_barrier()

    rows_per_w = NROWS_E // (NC * NS)
    r0 = (cc * NS + ss) * rows_per_w

    def body(g, _):
        pltpu.sync_copy(dst2d.at[pl.ds(r0 + g * 4, 4)], idx_v)
        for j in range(4):
            pltpu.sync_copy(ones_v, dacc.at[idx_v.at[j]], add=True)
        return 0

    lax.fori_loop(0, rows_per_w // 4, body, 0)
    plsc.subcore_barrier()
    pltpu.sync_copy(dacc.at[pl.ds(ss * TPW, TPW)], zbuf)
    pltpu.sync_copy(zbuf, out.at[pl.ds(cc * NPAD + ss * TPW, TPW)])


@functools.partial(
    pl.kernel,
    mesh=_mesh,
    out_type=jax.ShapeDtypeStruct((NC, NPAD, HALF), jnp.float32),
    scratch_types=[
        pltpu.VMEM((SB, K), jnp.int32),
        pltpu.VMEM((SB, K), jnp.int32),
        pltpu.VMEM((K, HALF), jnp.float32),
        pltpu.VMEM((TPW // 8, HALF), jnp.float32),
        pltpu.VMEM_SHARED((NPAD, HALF), jnp.float32),
        pltpu.SemaphoreType.DMA,
    ],
    compiler_params=pltpu.CompilerParams(use_tc_tiling_on_sc=False),
)
def _edge_kernel(lo, hi, src2d, dst2d, out, src_v, dst_v, rows_v, zbuf, acc,
                 sem):
    cc = lax.axis_index("c")
    ss = lax.axis_index("s")
    z16 = jnp.zeros((16,), jnp.float32)

    zrows = TPW // 8

    def initz(r, _):
        zbuf[r, pl.ds(0, 16)] = z16
        zbuf[r, pl.ds(16, 16)] = z16
        return 0

    lax.fori_loop(0, zrows, initz, 0)
    for j in range(8):
        pltpu.sync_copy(zbuf, acc.at[pl.ds(ss * TPW + j * zrows, zrows)])
    plsc.subcore_barrier()

    rows_per_t = NROWS_E // NS

    def make_body(tbl):
        def body(g, _):
            r0 = ss * rows_per_t + g * SB
            pltpu.sync_copy(src2d.at[pl.ds(r0, SB)], src_v)
            pltpu.sync_copy(dst2d.at[pl.ds(r0, SB)], dst_v)
            for j in range(SB):
                pltpu.async_copy(tbl.at[src_v.at[j]], rows_v, sem).wait()
                pltpu.sync_copy(rows_v, acc.at[dst_v.at[j]], add=True)
            return 0
        return body

    @pl.when(cc == 0)
    def _():
        lax.fori_loop(0, rows_per_t // SB, make_body(lo), 0)

    @pl.when(cc == 1)
    def _():
        lax.fori_loop(0, rows_per_t // SB, make_body(hi), 0)

    plsc.subcore_barrier()
    for j in range(8):
        pltpu.sync_copy(acc.at[pl.ds(ss * TPW + j * zrows, zrows)], zbuf)
        pltpu.sync_copy(zbuf, out.at[cc, pl.ds(ss * TPW + j * zrows, zrows)])


def _dinv_from(dp):
    deg = 1.0 + dp[0, 0, 0] + dp[1, 0, 0]
    return lax.rsqrt(deg)


def _dense1_body(x_ref, dp_ref, w1_ref, xw_ref, lo_ref, hi_ref):
    dinv = _dinv_from(dp_ref[...])
    xw = jnp.dot(x_ref[...], w1_ref[...], preferred_element_type=jnp.float32)
    xws = xw * dinv[:, None]
    xw_ref[...] = xw
    lo_ref[...] = xws[:, :HALF]
    hi_ref[...] = xws[:, HALF:]


def _dense2_body(acc_ref, xw_ref, dp_ref, b1_ref, w2_ref,
                 xw2_ref, lo_ref, hi_ref):
    dinv = _dinv_from(dp_ref[...])
    xw = xw_ref[...]
    agg = jnp.concatenate([acc_ref[0], acc_ref[1]], axis=1)
    h = dinv[:, None] * agg + (dinv * dinv)[:, None] * xw + b1_ref[...]
    h = jnp.maximum(h, 0.0)
    xw2 = jnp.dot(h, w2_ref[...], preferred_element_type=jnp.float32)
    xws2 = xw2 * dinv[:, None]
    xw2_ref[...] = xw2
    lo_ref[...] = xws2[:, :HALF]
    hi_ref[...] = xws2[:, HALF:]


def _head_body(acc_ref, xw_ref, dp_ref, b2_ref, batch_ref, wfc_ref, bfc_ref,
               out_ref, sums_ref, cnt_ref):
    i = pl.program_id(0)
    dinv = _dinv_from(dp_ref[...])
    xw = xw_ref[...]
    agg = jnp.concatenate([acc_ref[0], acc_ref[1]], axis=1)
    h = dinv[:, None] * agg + (dinv * dinv)[:, None] * xw + b2_ref[...]
    h = jnp.maximum(h, 0.0)

    b = batch_ref[0, 0]
    onehot = (b[:, None] == lax.broadcasted_iota(jnp.int32, (BLK, N_GRAPHS),
                                                 1)).astype(jnp.float32)

    @pl.when(i == 0)
    def _():
        sums_ref[...] = jnp.zeros_like(sums_ref)
        cnt_ref[...] = jnp.zeros_like(cnt_ref)

    sums_ref[...] += lax.dot_general(onehot, h, (((0,), (0,)), ((), ())),
                                     preferred_element_type=jnp.float32)
    cnt_ref[...] += jnp.sum(onehot, axis=0, keepdims=True)

    @pl.when(i == NBLK - 1)
    def _():
        g = sums_ref[...] / jnp.maximum(cnt_ref[...], 1.0).reshape(
            N_GRAPHS, 1)
        res = jnp.sum(g * wfc_ref[...], axis=1) + bfc_ref[0, 0]
        out_ref[...] = res.reshape(1, N_GRAPHS)


def _dense1(x, deg4, W1):
    return pl.pallas_call(
        _dense1_body,
        grid=(NBLK,),
        in_specs=[
            pl.BlockSpec((BLK, D_IN), lambda i: (i, 0)),
            pl.BlockSpec((2, 1, 1, BLK), lambda i: (0, i, 0, 0)),
            pl.BlockSpec((D_IN, D_H), lambda i: (0, 0)),
        ],
        out_specs=[
            pl.BlockSpec((BLK, D_H), lambda i: (i, 0)),
            pl.BlockSpec((BLK, HALF), lambda i: (i, 0)),
            pl.BlockSpec((BLK, HALF), lambda i: (i, 0)),
        ],
        out_shape=[
            jax.ShapeDtypeStruct((N, D_H), jnp.float32),
            jax.ShapeDtypeStruct((N, HALF), jnp.float32),
            jax.ShapeDtypeStruct((N, HALF), jnp.float32),
        ],
    )(x, deg4, W1)


def _dense2(acc, xw, deg4, b1, W2):
    return pl.pallas_call(
        _dense2_body,
        grid=(NBLK,),
        in_specs=[
            pl.BlockSpec((2, BLK, HALF), lambda i: (0, i, 0)),
            pl.BlockSpec((BLK, D_H), lambda i: (i, 0)),
            pl.BlockSpec((2, 1, 1, BLK), lambda i: (0, i, 0, 0)),
            pl.BlockSpec((1, D_H), lambda i: (0, 0)),
            pl.BlockSpec((D_H, D_H), lambda i: (0, 0)),
        ],
        out_specs=[
            pl.BlockSpec((BLK, D_H), lambda i: (i, 0)),
            pl.BlockSpec((BLK, HALF), lambda i: (i, 0)),
            pl.BlockSpec((BLK, HALF), lambda i: (i, 0)),
        ],
        out_shape=[
            jax.ShapeDtypeStruct((N, D_H), jnp.float32),
            jax.ShapeDtypeStruct((N, HALF), jnp.float32),
            jax.ShapeDtypeStruct((N, HALF), jnp.float32),
        ],
    )(acc, xw, deg4, b1, W2)


def _head(acc, xw, deg4, b2, batch3, wfc_row, bfc2):
    return pl.pallas_call(
        _head_body,
        grid=(NBLK,),
        in_specs=[
            pl.BlockSpec((2, BLK, HALF), lambda i: (0, i, 0)),
            pl.BlockSpec((BLK, D_H), lambda i: (i, 0)),
            pl.BlockSpec((2, 1, 1, BLK), lambda i: (0, i, 0, 0)),
            pl.BlockSpec((1, D_H), lambda i: (0, 0)),
            pl.BlockSpec((1, 1, BLK), lambda i: (i, 0, 0)),
            pl.BlockSpec((1, D_H), lambda i: (0, 0)),
            pl.BlockSpec((1, 1), lambda i: (0, 0)),
        ],
        out_specs=pl.BlockSpec((1, N_GRAPHS), lambda i: (0, 0)),
        out_shape=jax.ShapeDtypeStruct((1, N_GRAPHS), jnp.float32),
        scratch_shapes=[
            pltpu.VMEM((N_GRAPHS, D_H), jnp.float32),
            pltpu.VMEM((1, N_GRAPHS), jnp.float32),
        ],
    )(acc, xw, deg4, b2, batch3, wfc_row, bfc2)


def kernel(x, edge_index, batch, W1, b1, W2, b2, Wfc, bfc):
    src = edge_index[0]
    dst = edge_index[1]
    pad = E_PAD - E
    src_p = jnp.concatenate([src, jnp.zeros((pad,), jnp.int32)])
    dst_p = jnp.concatenate([dst, jnp.full((pad,), N, jnp.int32)])
    src2d = src_p.reshape(NROWS_E, K)
    dst2d = dst_p.reshape(NROWS_E, K)

    deg_p = _deg_kernel(dst2d).reshape(NC, NPAD)
    deg4 = deg_p[:, :N].reshape(2, NBLK, 1, BLK)

    batch3 = batch.reshape(NBLK, 1, BLK)
    b1r = b1.reshape(1, D_H)
    b2r = b2.reshape(1, D_H)
    wfc_row = Wfc.reshape(1, D_H)
    bfc2 = bfc.reshape(1, 1)

    xw1, lo1, hi1 = _dense1(x, deg4, W1)
    acc1 = _edge_kernel(lo1, hi1, src2d, dst2d)[:, :N]
    xw2, lo2, hi2 = _dense2(acc1, xw1, deg4, b1r, W2)
    acc2 = _edge_kernel(lo2, hi2, src2d, dst2d)[:, :N]
    out2d = _head(acc2, xw2, deg4, b2r, batch3, wfc_row, bfc2)
    return out2d.reshape(N_GRAPHS)


# speedup vs baseline: 25.3105x; 1.4026x over previous
import functools

import jax
import jax.numpy as jnp
from jax import lax
from jax.experimental import pallas as pl
from jax.experimental.pallas import tpu as pltpu
from jax.experimental.pallas import tpu_sc as plsc

N = 50000
E = 800000
D_IN = 128
D_H = 64
HALF = D_H // 2
N_GRAPHS = 64

NC = 2
NS = 16

K = 128
SB = 8
E_PAD = ((E + NC * NS * K - 1) // (NC * NS * K)) * (NC * NS * K)
NROWS_E = E_PAD // K
NPAD = 50176
TPW = NPAD // NS

BLK = 2000
NBLK = N // BLK

_mesh = plsc.VectorSubcoreMesh(core_axis_name="c", subcore_axis_name="s")


@functools.partial(
    pl.kernel,
    mesh=_mesh,
    out_type=jax.ShapeDtypeStruct((NC * NPAD,), jnp.float32),
    scratch_types=[
        pltpu.VMEM((4, K), jnp.int32),
        pltpu.VMEM((K,), jnp.float32),
        pltpu.VMEM((TPW,), jnp.float32),
        pltpu.VMEM_SHARED((NPAD,), jnp.float32),
    ],
)
def _deg_kernel(dst2d, out, idx_v, ones_v, zbuf, dacc):
    cc = lax.axis_index("c")
    ss = lax.axis_index("s")
    z16 = jnp.zeros((16,), jnp.float32)
    o16 = jnp.ones((16,), jnp.float32)

    def initz(i, _):
        zbuf[pl.ds(i * 16, 16)] = z16
        return 0

    lax.fori_loop(0, TPW // 16, initz, 0)
    for i in range(K // 16):
        ones_v[pl.ds(i * 16, 16)] = o16

    pltpu.sync_copy(zbuf, dacc.at[pl.ds(ss * TPW, TPW)])
    plsc.subcore_barrier()

    rows_per_w = NROWS_E // (NC * NS)
    r0 = (cc * NS + ss) * rows_per_w

    def body(g, _):
        pltpu.sync_copy(dst2d.at[pl.ds(r0 + g * 4, 4)], idx_v)
        for j in range(4):
            pltpu.sync_copy(ones_v, dacc.at[idx_v.at[j]], add=True)
        return 0

    lax.fori_loop(0, rows_per_w // 4, body, 0)
    plsc.subcore_barrier()
    pltpu.sync_copy(dacc.at[pl.ds(ss * TPW, TPW)], zbuf)
    pltpu.sync_copy(zbuf, out.at[pl.ds(cc * NPAD + ss * TPW, TPW)])


ROWS_PER_T = NROWS_E // NS
SBC = 4
NSUP = ROWS_PER_T // SBC


@functools.partial(
    pl.kernel,
    mesh=_mesh,
    out_type=jax.ShapeDtypeStruct((NC, NPAD, HALF), jnp.float32),
    scratch_types=[
        pltpu.VMEM((SBC, K), jnp.int32),
        pltpu.VMEM((SBC, K), jnp.int32),
        pltpu.VMEM((SBC, K), jnp.int32),
        pltpu.VMEM((SBC, K), jnp.int32),
        pltpu.VMEM((K, HALF), jnp.float32),
        pltpu.VMEM((K, HALF), jnp.float32),
        pltpu.VMEM((TPW // 8, HALF), jnp.float32),
        pltpu.VMEM_SHARED((NPAD, HALF), jnp.float32),
        pltpu.SemaphoreType.DMA,
        pltpu.SemaphoreType.DMA,
        pltpu.SemaphoreType.DMA,
        pltpu.SemaphoreType.DMA,
    ],
    compiler_params=pltpu.CompilerParams(use_tc_tiling_on_sc=False),
)
def _edge_kernel(lo, hi, src2d, dst2d, out, s0v, d0v, s1v, d1v, rows0, rows1,
                 zbuf, acc, sem_i0, sem_i1, sem_g0, sem_g1):
    cc = lax.axis_index("c")
    ss = lax.axis_index("s")
    z16 = jnp.zeros((16,), jnp.float32)
    rows = (rows0, rows1)
    sem_g = (sem_g0, sem_g1)
    sem_i = (sem_i0, sem_i1)

    def fire_idx(sup, ibs, ibd, sem):
        rr = ss * ROWS_PER_T + sup * SBC
        pltpu.async_copy(src2d.at[pl.ds(rr, SBC)], ibs, sem)
        pltpu.async_copy(dst2d.at[pl.ds(rr, SBC)], ibd, sem)

    def wait_idx(ibs, ibd, sem):
        pltpu.make_async_copy(src2d.at[pl.ds(0, SBC)], ibs, sem).wait()
        pltpu.make_async_copy(dst2d.at[pl.ds(0, SBC)], ibd, sem).wait()

    fire_idx(0, s0v, d0v, sem_i[0])

    zrows = TPW // 8

    def initz(r, _):
        zbuf[r, pl.ds(0, 16)] = z16
        zbuf[r, pl.ds(16, 16)] = z16
        return 0

    lax.fori_loop(0, zrows, initz, 0)
    for j in range(8):
        pltpu.sync_copy(zbuf, acc.at[pl.ds(ss * TPW + j * zrows, zrows)])

    wait_idx(s0v, d0v, sem_i[0])
    fire_idx(1, s1v, d1v, sem_i[1])
    plsc.subcore_barrier()

    def run(tbl):
        def fire_g(ib, j, b):
            pltpu.async_copy(tbl.at[ib.at[j]], rows[b], sem_g[b])

        def drain_g(b):
            pltpu.make_async_copy(tbl.at[s0v.at[0]], rows[b], sem_g[b]).wait()

        def phase(ibs, ibd):
            for j in range(SBC):
                if j + 1 < SBC:
                    fire_g(ibs, j + 1, (j + 1) % 2)
                drain_g(j % 2)
                pltpu.sync_copy(rows[j % 2], acc.at[ibd.at[j]], add=True)

        fire_g(s0v, 0, 0)

        def body(t, _):
            sup0 = 2 * t
            phase(s0v, d0v)

            @pl.when(t < NSUP // 2 - 1)
            def _():
                fire_idx(sup0 + 2, s0v, d0v, sem_i[0])

            wait_idx(s1v, d1v, sem_i[1])
            fire_g(s1v, 0, 0)
            phase(s1v, d1v)

            @pl.when(t < NSUP // 2 - 1)
            def _():
                fire_idx(sup0 + 3, s1v, d1v, sem_i[1])
                wait_idx(s0v, d0v, sem_i[0])
                fire_g(s0v, 0, 0)

            return 0

        lax.fori_loop(0, NSUP // 2, body, 0)

    @pl.when(cc == 0)
    def _():
        run(lo)

    @pl.when(cc == 1)
    def _():
        run(hi)

    plsc.subcore_barrier()
    for j in range(8):
        pltpu.sync_copy(acc.at[pl.ds(ss * TPW + j * zrows, zrows)], zbuf)
        pltpu.sync_copy(zbuf, out.at[cc, pl.ds(ss * TPW + j * zrows, zrows)])


def _dinv_from(dp):
    deg = 1.0 + dp[0, 0, 0] + dp[1, 0, 0]
    return lax.rsqrt(deg)


def _dense1_body(x_ref, dp_ref, w1_ref, xw_ref, lo_ref, hi_ref):
    dinv = _dinv_from(dp_ref[...])
    xw = jnp.dot(x_ref[...], w1_ref[...], preferred_element_type=jnp.float32)
    xws = xw * dinv[:, None]
    xw_ref[...] = xw
    lo_ref[...] = xws[:, :HALF]
    hi_ref[...] = xws[:, HALF:]


def _dense2_body(acc_ref, xw_ref, dp_ref, b1_ref, w2_ref,
                 xw2_ref, lo_ref, hi_ref):
    dinv = _dinv_from(dp_ref[...])
    xw = xw_ref[...]
    agg = jnp.concatenate([acc_ref[0], acc_ref[1]], axis=1)
    h = dinv[:, None] * agg + (dinv * dinv)[:, None] * xw + b1_ref[...]
    h = jnp.maximum(h, 0.0)
    xw2 = jnp.dot(h, w2_ref[...], preferred_element_type=jnp.float32)
    xws2 = xw2 * dinv[:, None]
    xw2_ref[...] = xw2
    lo_ref[...] = xws2[:, :HALF]
    hi_ref[...] = xws2[:, HALF:]


def _head_body(acc_ref, xw_ref, dp_ref, b2_ref, batch_ref, wfc_ref, bfc_ref,
               out_ref, sums_ref, cnt_ref):
    i = pl.program_id(0)
    dinv = _dinv_from(dp_ref[...])
    xw = xw_ref[...]
    agg = jnp.concatenate([acc_ref[0], acc_ref[1]], axis=1)
    h = dinv[:, None] * agg + (dinv * dinv)[:, None] * xw + b2_ref[...]
    h = jnp.maximum(h, 0.0)

    b = batch_ref[0, 0]
    onehot = (b[:, None] == lax.broadcasted_iota(jnp.int32, (BLK, N_GRAPHS),
                                                 1)).astype(jnp.float32)

    @pl.when(i == 0)
    def _():
        sums_ref[...] = jnp.zeros_like(sums_ref)
        cnt_ref[...] = jnp.zeros_like(cnt_ref)

    sums_ref[...] += lax.dot_general(onehot, h, (((0,), (0,)), ((), ())),
                                     preferred_element_type=jnp.float32)
    cnt_ref[...] += jnp.sum(onehot, axis=0, keepdims=True)

    @pl.when(i == NBLK - 1)
    def _():
        g = sums_ref[...] / jnp.maximum(cnt_ref[...], 1.0).reshape(
            N_GRAPHS, 1)
        res = jnp.sum(g * wfc_ref[...], axis=1) + bfc_ref[0, 0]
        out_ref[...] = res.reshape(1, N_GRAPHS)


def _dense1(x, deg4, W1):
    return pl.pallas_call(
        _dense1_body,
        grid=(NBLK,),
        in_specs=[
            pl.BlockSpec((BLK, D_IN), lambda i: (i, 0)),
            pl.BlockSpec((2, 1, 1, BLK), lambda i: (0, i, 0, 0)),
            pl.BlockSpec((D_IN, D_H), lambda i: (0, 0)),
        ],
        out_specs=[
            pl.BlockSpec((BLK, D_H), lambda i: (i, 0)),
            pl.BlockSpec((BLK, HALF), lambda i: (i, 0)),
            pl.BlockSpec((BLK, HALF), lambda i: (i, 0)),
        ],
        out_shape=[
            jax.ShapeDtypeStruct((N, D_H), jnp.float32),
            jax.ShapeDtypeStruct((N, HALF), jnp.float32),
            jax.ShapeDtypeStruct((N, HALF), jnp.float32),
        ],
    )(x, deg4, W1)


def _dense2(acc, xw, deg4, b1, W2):
    return pl.pallas_call(
        _dense2_body,
        grid=(NBLK,),
        in_specs=[
            pl.BlockSpec((2, BLK, HALF), lambda i: (0, i, 0)),
            pl.BlockSpec((BLK, D_H), lambda i: (i, 0)),
            pl.BlockSpec((2, 1, 1, BLK), lambda i: (0, i, 0, 0)),
            pl.BlockSpec((1, D_H), lambda i: (0, 0)),
            pl.BlockSpec((D_H, D_H), lambda i: (0, 0)),
        ],
        out_specs=[
            pl.BlockSpec((BLK, D_H), lambda i: (i, 0)),
            pl.BlockSpec((BLK, HALF), lambda i: (i, 0)),
            pl.BlockSpec((BLK, HALF), lambda i: (i, 0)),
        ],
        out_shape=[
            jax.ShapeDtypeStruct((N, D_H), jnp.float32),
            jax.ShapeDtypeStruct((N, HALF), jnp.float32),
            jax.ShapeDtypeStruct((N, HALF), jnp.float32),
        ],
    )(acc, xw, deg4, b1, W2)


def _head(acc, xw, deg4, b2, batch3, wfc_row, bfc2):
    return pl.pallas_call(
        _head_body,
        grid=(NBLK,),
        in_specs=[
            pl.BlockSpec((2, BLK, HALF), lambda i: (0, i, 0)),
            pl.BlockSpec((BLK, D_H), lambda i: (i, 0)),
            pl.BlockSpec((2, 1, 1, BLK), lambda i: (0, i, 0, 0)),
            pl.BlockSpec((1, D_H), lambda i: (0, 0)),
            pl.BlockSpec((1, 1, BLK), lambda i: (i, 0, 0)),
            pl.BlockSpec((1, D_H), lambda i: (0, 0)),
            pl.BlockSpec((1, 1), lambda i: (0, 0)),
        ],
        out_specs=pl.BlockSpec((1, N_GRAPHS), lambda i: (0, 0)),
        out_shape=jax.ShapeDtypeStruct((1, N_GRAPHS), jnp.float32),
        scratch_shapes=[
            pltpu.VMEM((N_GRAPHS, D_H), jnp.float32),
            pltpu.VMEM((1, N_GRAPHS), jnp.float32),
        ],
    )(acc, xw, deg4, b2, batch3, wfc_row, bfc2)


def kernel(x, edge_index, batch, W1, b1, W2, b2, Wfc, bfc):
    src = edge_index[0]
    dst = edge_index[1]
    pad = E_PAD - E
    src_p = jnp.concatenate([src, jnp.zeros((pad,), jnp.int32)])
    dst_p = jnp.concatenate([dst, jnp.full((pad,), N, jnp.int32)])
    src2d = src_p.reshape(NROWS_E, K)
    dst2d = dst_p.reshape(NROWS_E, K)

    deg_p = _deg_kernel(dst2d).reshape(NC, NPAD)
    deg4 = deg_p[:, :N].reshape(2, NBLK, 1, BLK)

    batch3 = batch.reshape(NBLK, 1, BLK)
    b1r = b1.reshape(1, D_H)
    b2r = b2.reshape(1, D_H)
    wfc_row = Wfc.reshape(1, D_H)
    bfc2 = bfc.reshape(1, 1)

    xw1, lo1, hi1 = _dense1(x, deg4, W1)
    acc1 = _edge_kernel(lo1, hi1, src2d, dst2d)[:, :N]
    xw2, lo2, hi2 = _dense2(acc1, xw1, deg4, b1r, W2)
    acc2 = _edge_kernel(lo2, hi2, src2d, dst2d)[:, :N]
    out2d = _head(acc2, xw2, deg4, b2r, batch3, wfc_row, bfc2)
    return out2d.reshape(N_GRAPHS)


# speedup vs baseline: 31.7958x; 1.2562x over previous
import functools

import jax
import jax.numpy as jnp
from jax import lax
from jax.experimental import pallas as pl
from jax.experimental.pallas import tpu as pltpu
from jax.experimental.pallas import tpu_sc as plsc

N = 50000
E = 800000
D_IN = 128
D_H = 64
HALF = D_H // 2
N_GRAPHS = 64

NC = 2
NS = 16

K = 128
SB = 8
E_PAD = ((E + NC * NS * K - 1) // (NC * NS * K)) * (NC * NS * K)
NROWS_E = E_PAD // K
NPAD = 50176
TPW = NPAD // NS

BLK = 2000
NBLK = N // BLK

_mesh = plsc.VectorSubcoreMesh(core_axis_name="c", subcore_axis_name="s")


@functools.partial(
    pl.kernel,
    mesh=_mesh,
    out_type=jax.ShapeDtypeStruct((NC * NPAD,), jnp.float32),
    scratch_types=[
        pltpu.VMEM((4, K), jnp.int32),
        pltpu.VMEM((K,), jnp.float32),
        pltpu.VMEM((TPW,), jnp.float32),
        pltpu.VMEM_SHARED((NPAD,), jnp.float32),
    ],
)
def _deg_kernel(dst2d, out, idx_v, ones_v, zbuf, dacc):
    cc = lax.axis_index("c")
    ss = lax.axis_index("s")
    z16 = jnp.zeros((16,), jnp.float32)
    o16 = jnp.ones((16,), jnp.float32)

    def initz(i, _):
        zbuf[pl.ds(i * 16, 16)] = z16
        return 0

    lax.fori_loop(0, TPW // 16, initz, 0)
    for i in range(K // 16):
        ones_v[pl.ds(i * 16, 16)] = o16

    pltpu.sync_copy(zbuf, dacc.at[pl.ds(ss * TPW, TPW)])
    plsc.subcore_barrier()

    rows_per_w = NROWS_E // (NC * NS)
    r0 = (cc * NS + ss) * rows_per_w

    def body(g, _):
        pltpu.sync_copy(dst2d.at[pl.ds(r0 + g * 4, 4)], idx_v)
        for j in range(4):
            pltpu.sync_copy(ones_v, dacc.at[idx_v.at[j]], add=True)
        return 0

    lax.fori_loop(0, rows_per_w // 4, body, 0)
    plsc.subcore_barrier()
    pltpu.sync_copy(dacc.at[pl.ds(ss * TPW, TPW)], zbuf)
    pltpu.sync_copy(zbuf, out.at[pl.ds(cc * NPAD + ss * TPW, TPW)])


ROWS_PER_T = NROWS_E // NS
SBC = 4
NSUP = ROWS_PER_T // SBC


@functools.partial(
    pl.kernel,
    mesh=_mesh,
    out_type=jax.ShapeDtypeStruct((NC, NPAD, HALF), jnp.float32),
    scratch_types=[
        pltpu.VMEM((SBC, K), jnp.int32),
        pltpu.VMEM((SBC, K), jnp.int32),
        pltpu.VMEM((SBC, K), jnp.int32),
        pltpu.VMEM((SBC, K), jnp.int32),
        pltpu.VMEM((K, HALF), jnp.float32),
        pltpu.VMEM((K, HALF), jnp.float32),
        pltpu.VMEM((K, HALF), jnp.float32),
        pltpu.VMEM((K, HALF), jnp.float32),
        pltpu.VMEM((TPW // 16, HALF), jnp.float32),
        pltpu.VMEM_SHARED((NPAD, HALF), jnp.float32),
        pltpu.SemaphoreType.DMA,
        pltpu.SemaphoreType.DMA,
        pltpu.SemaphoreType.DMA,
        pltpu.SemaphoreType.DMA,
        pltpu.SemaphoreType.DMA,
        pltpu.SemaphoreType.DMA,
        pltpu.SemaphoreType.DMA,
        pltpu.SemaphoreType.DMA,
        pltpu.SemaphoreType.DMA,
        pltpu.SemaphoreType.DMA,
    ],
    compiler_params=pltpu.CompilerParams(use_tc_tiling_on_sc=False),
)
def _edge_kernel(lo, hi, src2d, dst2d, out, s0v, d0v, s1v, d1v,
                 rows0, rows1, rows2, rows3, zbuf, acc,
                 sem_i0, sem_i1, sem_g0, sem_g1, sem_g2, sem_g3,
                 sem_s0, sem_s1, sem_s2, sem_s3):
    cc = lax.axis_index("c")
    ss = lax.axis_index("s")
    z16 = jnp.zeros((16,), jnp.float32)
    rows = (rows0, rows1, rows2, rows3)
    sem_g = (sem_g0, sem_g1, sem_g2, sem_g3)
    sem_s = (sem_s0, sem_s1, sem_s2, sem_s3)
    sem_i = (sem_i0, sem_i1)

    def fire_idx(sup, ibs, ibd, sem, pred=None):
        def do():
            rr = ss * ROWS_PER_T + sup * SBC
            pltpu.async_copy(src2d.at[pl.ds(rr, SBC)], ibs, sem)
            pltpu.async_copy(dst2d.at[pl.ds(rr, SBC)], ibd, sem)
        if pred is None:
            do()
        else:
            pl.when(pred)(do)

    def wait_idx(ibs, ibd, sem, pred=None):
        def do():
            pltpu.make_async_copy(src2d.at[pl.ds(0, SBC)], ibs, sem).wait()
            pltpu.make_async_copy(dst2d.at[pl.ds(0, SBC)], ibd, sem).wait()
        if pred is None:
            do()
        else:
            pl.when(pred)(do)

    fire_idx(0, s0v, d0v, sem_i[0])

    zrows = TPW // 16

    def initz(r, _):
        zbuf[r, pl.ds(0, 16)] = z16
        zbuf[r, pl.ds(16, 16)] = z16
        return 0

    lax.fori_loop(0, zrows, initz, 0)
    for j in range(16):
        pltpu.sync_copy(zbuf, acc.at[pl.ds(ss * TPW + j * zrows, zrows)])

    wait_idx(s0v, d0v, sem_i[0])
    fire_idx(1, s1v, d1v, sem_i[1])
    plsc.subcore_barrier()

    def run(tbl):
        def fire_g(ib, j, b, pred=None):
            def do():
                pltpu.async_copy(tbl.at[ib.at[j]], rows[b], sem_g[b])
            if pred is None:
                do()
            else:
                pl.when(pred)(do)

        def drain_g(b):
            pltpu.make_async_copy(tbl.at[s0v.at[0]], rows[b], sem_g[b]).wait()

        def fire_s(ibd, j, b):
            pltpu.async_copy(rows[b], acc.at[ibd.at[j]], sem_s[b], add=True)

        def drain_s(b, pred=None):
            def do():
                pltpu.make_async_copy(rows[b], acc.at[d0v.at[0]],
                                      sem_s[b]).wait()
            if pred is None:
                do()
            else:
                pl.when(pred)(do)

        fire_g(s0v, 0, 0)
        fire_g(s0v, 1, 1)

        def body(t, _):
            not_first = t > 0
            not_last = t < NSUP // 2 - 1
            drain_s(2, pred=not_first)
            fire_g(s0v, 2, 2)
            drain_g(0)
            fire_s(d0v, 0, 0)
            drain_s(3, pred=not_first)
            fire_g(s0v, 3, 3)
            drain_g(1)
            fire_s(d0v, 1, 1)
            wait_idx(s1v, d1v, sem_i[1])
            drain_s(0)
            fire_g(s1v, 0, 0)
            drain_g(2)
            fire_s(d0v, 2, 2)
            drain_s(1)
            fire_g(s1v, 1, 1)
            drain_g(3)
            fire_s(d0v, 3, 3)
            fire_idx(2 * t + 2, s0v, d0v, sem_i[0], pred=not_last)
            drain_s(2)
            fire_g(s1v, 2, 2)
            drain_g(0)
            fire_s(d1v, 0, 0)
            drain_s(3)
            fire_g(s1v, 3, 3)
            drain_g(1)
            fire_s(d1v, 1, 1)
            wait_idx(s0v, d0v, sem_i[0], pred=not_last)
            drain_s(0)
            fire_g(s0v, 0, 0, pred=not_last)
            drain_g(2)
            fire_s(d1v, 2, 2)
            drain_s(1)
            fire_g(s0v, 1, 1, pred=not_last)
            drain_g(3)
            fire_s(d1v, 3, 3)
            fire_idx(2 * t + 3, s1v, d1v, sem_i[1], pred=not_last)
            return 0

        lax.fori_loop(0, NSUP // 2, body, 0)
        drain_s(2)
        drain_s(3)

    @pl.when(cc == 0)
    def _():
        run(lo)

    @pl.when(cc == 1)
    def _():
        run(hi)

    plsc.subcore_barrier()
    for j in range(16):
        pltpu.sync_copy(acc.at[pl.ds(ss * TPW + j * zrows, zrows)], zbuf)
        pltpu.sync_copy(zbuf, out.at[cc, pl.ds(ss * TPW + j * zrows, zrows)])


def _dinv_from(dp):
    deg = 1.0 + dp[0, 0, 0] + dp[1, 0, 0]
    return lax.rsqrt(deg)


def _dense1_body(x_ref, dp_ref, w1_ref, xw_ref, lo_ref, hi_ref):
    dinv = _dinv_from(dp_ref[...])
    xw = jnp.dot(x_ref[...], w1_ref[...], preferred_element_type=jnp.float32)
    xws = xw * dinv[:, None]
    xw_ref[...] = xw
    lo_ref[...] = xws[:, :HALF]
    hi_ref[...] = xws[:, HALF:]


def _dense2_body(acc_ref, xw_ref, dp_ref, b1_ref, w2_ref,
                 xw2_ref, lo_ref, hi_ref):
    dinv = _dinv_from(dp_ref[...])
    xw = xw_ref[...]
    agg = jnp.concatenate([acc_ref[0], acc_ref[1]], axis=1)
    h = dinv[:, None] * agg + (dinv * dinv)[:, None] * xw + b1_ref[...]
    h = jnp.maximum(h, 0.0)
    xw2 = jnp.dot(h, w2_ref[...], preferred_element_type=jnp.float32)
    xws2 = xw2 * dinv[:, None]
    xw2_ref[...] = xw2
    lo_ref[...] = xws2[:, :HALF]
    hi_ref[...] = xws2[:, HALF:]


def _head_body(acc_ref, xw_ref, dp_ref, b2_ref, batch_ref, wfc_ref, bfc_ref,
               out_ref, sums_ref, cnt_ref):
    i = pl.program_id(0)
    dinv = _dinv_from(dp_ref[...])
    xw = xw_ref[...]
    agg = jnp.concatenate([acc_ref[0], acc_ref[1]], axis=1)
    h = dinv[:, None] * agg + (dinv * dinv)[:, None] * xw + b2_ref[...]
    h = jnp.maximum(h, 0.0)

    b = batch_ref[0, 0]
    onehot = (b[:, None] == lax.broadcasted_iota(jnp.int32, (BLK, N_GRAPHS),
                                                 1)).astype(jnp.float32)

    @pl.when(i == 0)
    def _():
        sums_ref[...] = jnp.zeros_like(sums_ref)
        cnt_ref[...] = jnp.zeros_like(cnt_ref)

    sums_ref[...] += lax.dot_general(onehot, h, (((0,), (0,)), ((), ())),
                                     preferred_element_type=jnp.float32)
    cnt_ref[...] += jnp.sum(onehot, axis=0, keepdims=True)

    @pl.when(i == NBLK - 1)
    def _():
        g = sums_ref[...] / jnp.maximum(cnt_ref[...], 1.0).reshape(
            N_GRAPHS, 1)
        res = jnp.sum(g * wfc_ref[...], axis=1) + bfc_ref[0, 0]
        out_ref[...] = res.reshape(1, N_GRAPHS)


def _dense1(x, deg4, W1):
    return pl.pallas_call(
        _dense1_body,
        grid=(NBLK,),
        in_specs=[
            pl.BlockSpec((BLK, D_IN), lambda i: (i, 0)),
            pl.BlockSpec((2, 1, 1, BLK), lambda i: (0, i, 0, 0)),
            pl.BlockSpec((D_IN, D_H), lambda i: (0, 0)),
        ],
        out_specs=[
            pl.BlockSpec((BLK, D_H), lambda i: (i, 0)),
            pl.BlockSpec((BLK, HALF), lambda i: (i, 0)),
            pl.BlockSpec((BLK, HALF), lambda i: (i, 0)),
        ],
        out_shape=[
            jax.ShapeDtypeStruct((N, D_H), jnp.float32),
            jax.ShapeDtypeStruct((N, HALF), jnp.float32),
            jax.ShapeDtypeStruct((N, HALF), jnp.float32),
        ],
    )(x, deg4, W1)


def _dense2(acc, xw, deg4, b1, W2):
    return pl.pallas_call(
        _dense2_body,
        grid=(NBLK,),
        in_specs=[
            pl.BlockSpec((2, BLK, HALF), lambda i: (0, i, 0)),
            pl.BlockSpec((BLK, D_H), lambda i: (i, 0)),
            pl.BlockSpec((2, 1, 1, BLK), lambda i: (0, i, 0, 0)),
            pl.BlockSpec((1, D_H), lambda i: (0, 0)),
            pl.BlockSpec((D_H, D_H), lambda i: (0, 0)),
        ],
        out_specs=[
            pl.BlockSpec((BLK, D_H), lambda i: (i, 0)),
            pl.BlockSpec((BLK, HALF), lambda i: (i, 0)),
            pl.BlockSpec((BLK, HALF), lambda i: (i, 0)),
        ],
        out_shape=[
            jax.ShapeDtypeStruct((N, D_H), jnp.float32),
            jax.ShapeDtypeStruct((N, HALF), jnp.float32),
            jax.ShapeDtypeStruct((N, HALF), jnp.float32),
        ],
    )(acc, xw, deg4, b1, W2)


def _head(acc, xw, deg4, b2, batch3, wfc_row, bfc2):
    return pl.pallas_call(
        _head_body,
        grid=(NBLK,),
        in_specs=[
            pl.BlockSpec((2, BLK, HALF), lambda i: (0, i, 0)),
            pl.BlockSpec((BLK, D_H), lambda i: (i, 0)),
            pl.BlockSpec((2, 1, 1, BLK), lambda i: (0, i, 0, 0)),
            pl.BlockSpec((1, D_H), lambda i: (0, 0)),
            pl.BlockSpec((1, 1, BLK), lambda i: (i, 0, 0)),
            pl.BlockSpec((1, D_H), lambda i: (0, 0)),
            pl.BlockSpec((1, 1), lambda i: (0, 0)),
        ],
        out_specs=pl.BlockSpec((1, N_GRAPHS), lambda i: (0, 0)),
        out_shape=jax.ShapeDtypeStruct((1, N_GRAPHS), jnp.float32),
        scratch_shapes=[
            pltpu.VMEM((N_GRAPHS, D_H), jnp.float32),
            pltpu.VMEM((1, N_GRAPHS), jnp.float32),
        ],
    )(acc, xw, deg4, b2, batch3, wfc_row, bfc2)


def kernel(x, edge_index, batch, W1, b1, W2, b2, Wfc, bfc):
    src = edge_index[0]
    dst = edge_index[1]
    pad = E_PAD - E
    src_p = jnp.concatenate([src, jnp.zeros((pad,), jnp.int32)])
    dst_p = jnp.concatenate([dst, jnp.full((pad,), N, jnp.int32)])
    src2d = src_p.reshape(NROWS_E, K)
    dst2d = dst_p.reshape(NROWS_E, K)

    deg_p = _deg_kernel(dst2d).reshape(NC, NPAD)
    deg4 = deg_p[:, :N].reshape(2, NBLK, 1, BLK)

    batch3 = batch.reshape(NBLK, 1, BLK)
    b1r = b1.reshape(1, D_H)
    b2r = b2.reshape(1, D_H)
    wfc_row = Wfc.reshape(1, D_H)
    bfc2 = bfc.reshape(1, 1)

    xw1, lo1, hi1 = _dense1(x, deg4, W1)
    acc1 = _edge_kernel(lo1, hi1, src2d, dst2d)[:, :N]
    xw2, lo2, hi2 = _dense2(acc1, xw1, deg4, b1r, W2)
    acc2 = _edge_kernel(lo2, hi2, src2d, dst2d)[:, :N]
    out2d = _head(acc2, xw2, deg4, b2r, batch3, wfc_row, bfc2)
    return out2d.reshape(N_GRAPHS)


# speedup vs baseline: 35.5475x; 1.1180x over previous
import functools

import jax
import jax.numpy as jnp
from jax import lax
from jax.experimental import pallas as pl
from jax.experimental.pallas import tpu as pltpu
from jax.experimental.pallas import tpu_sc as plsc

N = 50000
E = 800000
D_IN = 128
D_H = 64
HALF = D_H // 2
N_GRAPHS = 64

NC = 2
NS = 16

K = 128
SB = 8
E_PAD = ((E + NC * NS * K - 1) // (NC * NS * K)) * (NC * NS * K)
NROWS_E = E_PAD // K
NPAD = 50176
TPW = NPAD // NS

BLK = 5000
NBLK = N // BLK

_mesh = plsc.VectorSubcoreMesh(core_axis_name="c", subcore_axis_name="s")


@functools.partial(
    pl.kernel,
    mesh=_mesh,
    out_type=jax.ShapeDtypeStruct((NC * NPAD,), jnp.float32),
    scratch_types=[
        pltpu.VMEM((4, K), jnp.int32),
        pltpu.VMEM((K,), jnp.float32),
        pltpu.VMEM((TPW,), jnp.float32),
        pltpu.VMEM_SHARED((NPAD,), jnp.float32),
    ],
)
def _deg_kernel(dst2d, out, idx_v, ones_v, zbuf, dacc):
    cc = lax.axis_index("c")
    ss = lax.axis_index("s")
    z16 = jnp.zeros((16,), jnp.float32)
    o16 = jnp.ones((16,), jnp.float32)

    def initz(i, _):
        zbuf[pl.ds(i * 16, 16)] = z16
        return 0

    lax.fori_loop(0, TPW // 16, initz, 0)
    for i in range(K // 16):
        ones_v[pl.ds(i * 16, 16)] = o16

    pltpu.sync_copy(zbuf, dacc.at[pl.ds(ss * TPW, TPW)])
    plsc.subcore_barrier()

    rows_per_w = NROWS_E // (NC * NS)
    r0 = (cc * NS + ss) * rows_per_w

    def body(g, _):
        pltpu.sync_copy(dst2d.at[pl.ds(r0 + g * 4, 4)], idx_v)
        for j in range(4):
            pltpu.sync_copy(ones_v, dacc.at[idx_v.at[j]], add=True)
        return 0

    lax.fori_loop(0, rows_per_w // 4, body, 0)
    plsc.subcore_barrier()
    pltpu.sync_copy(dacc.at[pl.ds(ss * TPW, TPW)], zbuf)
    pltpu.sync_copy(zbuf, out.at[pl.ds(cc * NPAD + ss * TPW, TPW)])


ROWS_PER_T = NROWS_E // NS
SBC = 4
NSUP = ROWS_PER_T // SBC


@functools.partial(
    pl.kernel,
    mesh=_mesh,
    out_type=jax.ShapeDtypeStruct((NC, NPAD, HALF), jnp.float32),
    scratch_types=[
        pltpu.VMEM((SBC, K), jnp.int32),
        pltpu.VMEM((SBC, K), jnp.int32),
        pltpu.VMEM((SBC, K), jnp.int32),
        pltpu.VMEM((SBC, K), jnp.int32),
        pltpu.VMEM((K, HALF), jnp.float32),
        pltpu.VMEM((K, HALF), jnp.float32),
        pltpu.VMEM((K, HALF), jnp.float32),
        pltpu.VMEM((K, HALF), jnp.float32),
        pltpu.VMEM((TPW // 16, HALF), jnp.float32),
        pltpu.VMEM_SHARED((NPAD, HALF), jnp.float32),
        pltpu.SemaphoreType.DMA,
        pltpu.SemaphoreType.DMA,
        pltpu.SemaphoreType.DMA,
        pltpu.SemaphoreType.DMA,
        pltpu.SemaphoreType.DMA,
        pltpu.SemaphoreType.DMA,
        pltpu.SemaphoreType.DMA,
        pltpu.SemaphoreType.DMA,
        pltpu.SemaphoreType.DMA,
        pltpu.SemaphoreType.DMA,
    ],
    compiler_params=pltpu.CompilerParams(use_tc_tiling_on_sc=False),
)
def _edge_kernel(lo, hi, src2d, dst2d, out, s0v, d0v, s1v, d1v,
                 rows0, rows1, rows2, rows3, zbuf, acc,
                 sem_i0, sem_i1, sem_g0, sem_g1, sem_g2, sem_g3,
                 sem_s0, sem_s1, sem_s2, sem_s3):
    cc = lax.axis_index("c")
    ss = lax.axis_index("s")
    z16 = jnp.zeros((16,), jnp.float32)
    rows = (rows0, rows1, rows2, rows3)
    sem_g = (sem_g0, sem_g1, sem_g2, sem_g3)
    sem_s = (sem_s0, sem_s1, sem_s2, sem_s3)
    sem_i = (sem_i0, sem_i1)

    def fire_idx(sup, ibs, ibd, sem, pred=None):
        def do():
            rr = ss * ROWS_PER_T + sup * SBC
            pltpu.async_copy(src2d.at[pl.ds(rr, SBC)], ibs, sem)
            pltpu.async_copy(dst2d.at[pl.ds(rr, SBC)], ibd, sem)
        if pred is None:
            do()
        else:
            pl.when(pred)(do)

    def wait_idx(ibs, ibd, sem, pred=None):
        def do():
            pltpu.make_async_copy(src2d.at[pl.ds(0, SBC)], ibs, sem).wait()
            pltpu.make_async_copy(dst2d.at[pl.ds(0, SBC)], ibd, sem).wait()
        if pred is None:
            do()
        else:
            pl.when(pred)(do)

    fire_idx(0, s0v, d0v, sem_i[0])

    zrows = TPW // 16

    def initz(r, _):
        zbuf[r, pl.ds(0, 16)] = z16
        zbuf[r, pl.ds(16, 16)] = z16
        return 0

    lax.fori_loop(0, zrows, initz, 0)
    for j in range(16):
        pltpu.sync_copy(zbuf, acc.at[pl.ds(ss * TPW + j * zrows, zrows)])

    wait_idx(s0v, d0v, sem_i[0])
    fire_idx(1, s1v, d1v, sem_i[1])
    plsc.subcore_barrier()

    def run(tbl):
        def fire_g(ib, j, b, pred=None):
            def do():
                pltpu.async_copy(tbl.at[ib.at[j]], rows[b], sem_g[b])
            if pred is None:
                do()
            else:
                pl.when(pred)(do)

        def drain_g(b):
            pltpu.make_async_copy(tbl.at[s0v.at[0]], rows[b], sem_g[b]).wait()

        def fire_s(ibd, j, b):
            pltpu.async_copy(rows[b], acc.at[ibd.at[j]], sem_s[b], add=True)

        def drain_s(b, pred=None):
            def do():
                pltpu.make_async_copy(rows[b], acc.at[d0v.at[0]],
                                      sem_s[b]).wait()
            if pred is None:
                do()
            else:
                pl.when(pred)(do)

        fire_g(s0v, 0, 0)
        fire_g(s0v, 1, 1)

        def body(t, _):
            not_first = t > 0
            not_last = t < NSUP // 2 - 1
            drain_s(2, pred=not_first)
            fire_g(s0v, 2, 2)
            drain_g(0)
            fire_s(d0v, 0, 0)
            drain_s(3, pred=not_first)
            fire_g(s0v, 3, 3)
            drain_g(1)
            fire_s(d0v, 1, 1)
            wait_idx(s1v, d1v, sem_i[1])
            drain_s(0)
            fire_g(s1v, 0, 0)
            drain_g(2)
            fire_s(d0v, 2, 2)
            drain_s(1)
            fire_g(s1v, 1, 1)
            drain_g(3)
            fire_s(d0v, 3, 3)
            fire_idx(2 * t + 2, s0v, d0v, sem_i[0], pred=not_last)
            drain_s(2)
            fire_g(s1v, 2, 2)
            drain_g(0)
            fire_s(d1v, 0, 0)
            drain_s(3)
            fire_g(s1v, 3, 3)
            drain_g(1)
            fire_s(d1v, 1, 1)
            wait_idx(s0v, d0v, sem_i[0], pred=not_last)
            drain_s(0)
            fire_g(s0v, 0, 0, pred=not_last)
            drain_g(2)
            fire_s(d1v, 2, 2)
            drain_s(1)
            fire_g(s0v, 1, 1, pred=not_last)
            drain_g(3)
            fire_s(d1v, 3, 3)
            fire_idx(2 * t + 3, s1v, d1v, sem_i[1], pred=not_last)
            return 0

        lax.fori_loop(0, NSUP // 2, body, 0)
        drain_s(2)
        drain_s(3)

    @pl.when(cc == 0)
    def _():
        run(lo)

    @pl.when(cc == 1)
    def _():
        run(hi)

    plsc.subcore_barrier()
    for j in range(16):
        pltpu.sync_copy(acc.at[pl.ds(ss * TPW + j * zrows, zrows)], zbuf)
        pltpu.sync_copy(zbuf, out.at[cc, pl.ds(ss * TPW + j * zrows, zrows)])


def _dinv_from(dp):
    deg = 1.0 + dp[0, 0, 0] + dp[1, 0, 0]
    return lax.rsqrt(deg)


def _dense1_body(x_ref, dp_ref, w1_ref, xw_ref, lo_ref, hi_ref):
    dinv = _dinv_from(dp_ref[...])
    xw = jnp.dot(x_ref[...], w1_ref[...], preferred_element_type=jnp.float32)
    xws = xw * dinv[:, None]
    xw_ref[...] = xw
    lo_ref[...] = xws[:, :HALF]
    hi_ref[...] = xws[:, HALF:]


def _dense2_body(acc_ref, xw_ref, dp_ref, b1_ref, w2_ref,
                 xw2_ref, lo_ref, hi_ref):
    dinv = _dinv_from(dp_ref[...])
    xw = xw_ref[...]
    agg = jnp.concatenate([acc_ref[0], acc_ref[1]], axis=1)
    h = dinv[:, None] * agg + (dinv * dinv)[:, None] * xw + b1_ref[...]
    h = jnp.maximum(h, 0.0)
    xw2 = jnp.dot(h, w2_ref[...], preferred_element_type=jnp.float32)
    xws2 = xw2 * dinv[:, None]
    xw2_ref[...] = xw2
    lo_ref[...] = xws2[:, :HALF]
    hi_ref[...] = xws2[:, HALF:]


def _head_body(acc_ref, xw_ref, dp_ref, b2_ref, batch_ref, wfc_ref, bfc_ref,
               out_ref, sums_ref, cnt_ref):
    i = pl.program_id(0)
    dinv = _dinv_from(dp_ref[...])
    xw = xw_ref[...]
    agg = jnp.concatenate([acc_ref[0], acc_ref[1]], axis=1)
    h = dinv[:, None] * agg + (dinv * dinv)[:, None] * xw + b2_ref[...]
    h = jnp.maximum(h, 0.0)

    b = batch_ref[0, 0]
    onehot = (b[:, None] == lax.broadcasted_iota(jnp.int32, (BLK, N_GRAPHS),
                                                 1)).astype(jnp.float32)

    @pl.when(i == 0)
    def _():
        sums_ref[...] = jnp.zeros_like(sums_ref)
        cnt_ref[...] = jnp.zeros_like(cnt_ref)

    sums_ref[...] += lax.dot_general(onehot, h, (((0,), (0,)), ((), ())),
                                     preferred_element_type=jnp.float32)
    cnt_ref[...] += jnp.sum(onehot, axis=0, keepdims=True)

    @pl.when(i == NBLK - 1)
    def _():
        g = sums_ref[...] / jnp.maximum(cnt_ref[...], 1.0).reshape(
            N_GRAPHS, 1)
        res = jnp.sum(g * wfc_ref[...], axis=1) + bfc_ref[0, 0]
        out_ref[...] = res.reshape(1, N_GRAPHS)


def _dense1(x, deg4, W1):
    return pl.pallas_call(
        _dense1_body,
        grid=(NBLK,),
        in_specs=[
            pl.BlockSpec((BLK, D_IN), lambda i: (i, 0)),
            pl.BlockSpec((2, 1, 1, BLK), lambda i: (0, i, 0, 0)),
            pl.BlockSpec((D_IN, D_H), lambda i: (0, 0)),
        ],
        out_specs=[
            pl.BlockSpec((BLK, D_H), lambda i: (i, 0)),
            pl.BlockSpec((BLK, HALF), lambda i: (i, 0)),
            pl.BlockSpec((BLK, HALF), lambda i: (i, 0)),
        ],
        out_shape=[
            jax.ShapeDtypeStruct((N, D_H), jnp.float32),
            jax.ShapeDtypeStruct((N, HALF), jnp.float32),
            jax.ShapeDtypeStruct((N, HALF), jnp.float32),
        ],
    )(x, deg4, W1)


def _dense2(acc, xw, deg4, b1, W2):
    return pl.pallas_call(
        _dense2_body,
        grid=(NBLK,),
        in_specs=[
            pl.BlockSpec((2, BLK, HALF), lambda i: (0, i, 0)),
            pl.BlockSpec((BLK, D_H), lambda i: (i, 0)),
            pl.BlockSpec((2, 1, 1, BLK), lambda i: (0, i, 0, 0)),
            pl.BlockSpec((1, D_H), lambda i: (0, 0)),
            pl.BlockSpec((D_H, D_H), lambda i: (0, 0)),
        ],
        out_specs=[
            pl.BlockSpec((BLK, D_H), lambda i: (i, 0)),
            pl.BlockSpec((BLK, HALF), lambda i: (i, 0)),
            pl.BlockSpec((BLK, HALF), lambda i: (i, 0)),
        ],
        out_shape=[
            jax.ShapeDtypeStruct((N, D_H), jnp.float32),
            jax.ShapeDtypeStruct((N, HALF), jnp.float32),
            jax.ShapeDtypeStruct((N, HALF), jnp.float32),
        ],
    )(acc, xw, deg4, b1, W2)


def _head(acc, xw, deg4, b2, batch3, wfc_row, bfc2):
    return pl.pallas_call(
        _head_body,
        grid=(NBLK,),
        in_specs=[
            pl.BlockSpec((2, BLK, HALF), lambda i: (0, i, 0)),
            pl.BlockSpec((BLK, D_H), lambda i: (i, 0)),
            pl.BlockSpec((2, 1, 1, BLK), lambda i: (0, i, 0, 0)),
            pl.BlockSpec((1, D_H), lambda i: (0, 0)),
            pl.BlockSpec((1, 1, BLK), lambda i: (i, 0, 0)),
            pl.BlockSpec((1, D_H), lambda i: (0, 0)),
            pl.BlockSpec((1, 1), lambda i: (0, 0)),
        ],
        out_specs=pl.BlockSpec((1, N_GRAPHS), lambda i: (0, 0)),
        out_shape=jax.ShapeDtypeStruct((1, N_GRAPHS), jnp.float32),
        scratch_shapes=[
            pltpu.VMEM((N_GRAPHS, D_H), jnp.float32),
            pltpu.VMEM((1, N_GRAPHS), jnp.float32),
        ],
    )(acc, xw, deg4, b2, batch3, wfc_row, bfc2)


def kernel(x, edge_index, batch, W1, b1, W2, b2, Wfc, bfc):
    src = edge_index[0]
    dst = edge_index[1]
    pad = E_PAD - E
    src_p = jnp.concatenate([src, jnp.zeros((pad,), jnp.int32)])
    dst_p = jnp.concatenate([dst, jnp.full((pad,), N, jnp.int32)])
    src2d = src_p.reshape(NROWS_E, K)
    dst2d = dst_p.reshape(NROWS_E, K)

    deg_p = _deg_kernel(dst2d).reshape(NC, NPAD)
    deg4 = deg_p[:, :N].reshape(2, NBLK, 1, BLK)

    batch3 = batch.reshape(NBLK, 1, BLK)
    b1r = b1.reshape(1, D_H)
    b2r = b2.reshape(1, D_H)
    wfc_row = Wfc.reshape(1, D_H)
    bfc2 = bfc.reshape(1, 1)

    xw1, lo1, hi1 = _dense1(x, deg4, W1)
    acc1 = _edge_kernel(lo1, hi1, src2d, dst2d)
    xw2, lo2, hi2 = _dense2(acc1, xw1, deg4, b1r, W2)
    acc2 = _edge_kernel(lo2, hi2, src2d, dst2d)
    out2d = _head(acc2, xw2, deg4, b2r, batch3, wfc_row, bfc2)
    return out2d.reshape(N_GRAPHS)


# speedup vs baseline: 35.6079x; 1.0017x over previous
import functools

import jax
import jax.numpy as jnp
from jax import lax
from jax.experimental import pallas as pl
from jax.experimental.pallas import tpu as pltpu
from jax.experimental.pallas import tpu_sc as plsc

N = 50000
E = 800000
D_IN = 128
D_H = 64
HALF = D_H // 2
N_GRAPHS = 64

NC = 2
NS = 16

K = 128
SB = 8
E_PAD = ((E + NC * NS * K - 1) // (NC * NS * K)) * (NC * NS * K)
NROWS_E = E_PAD // K
NPAD = 50176
TPW = NPAD // NS

BLK = 5000
NBLK = N // BLK

_mesh = plsc.VectorSubcoreMesh(core_axis_name="c", subcore_axis_name="s")


@functools.partial(
    pl.kernel,
    mesh=_mesh,
    out_type=jax.ShapeDtypeStruct((NC * NPAD,), jnp.float32),
    scratch_types=[
        pltpu.VMEM((4, K), jnp.int32),
        pltpu.VMEM((K,), jnp.float32),
        pltpu.VMEM((TPW,), jnp.float32),
        pltpu.VMEM_SHARED((NPAD,), jnp.float32),
    ],
)
def _deg_kernel(dst2d, out, idx_v, ones_v, zbuf, dacc):
    cc = lax.axis_index("c")
    ss = lax.axis_index("s")
    z16 = jnp.zeros((16,), jnp.float32)
    o16 = jnp.ones((16,), jnp.float32)

    def initz(i, _):
        zbuf[pl.ds(i * 16, 16)] = z16
        return 0

    lax.fori_loop(0, TPW // 16, initz, 0)
    for i in range(K // 16):
        ones_v[pl.ds(i * 16, 16)] = o16

    pltpu.sync_copy(zbuf, dacc.at[pl.ds(ss * TPW, TPW)])
    plsc.subcore_barrier()

    rows_per_w = NROWS_E // (NC * NS)
    r0 = (cc * NS + ss) * rows_per_w

    def body(g, _):
        pltpu.sync_copy(dst2d.at[pl.ds(r0 + g * 4, 4)], idx_v)
        for j in range(4):
            pltpu.sync_copy(ones_v, dacc.at[idx_v.at[j]], add=True)
        return 0

    lax.fori_loop(0, rows_per_w // 4, body, 0)
    plsc.subcore_barrier()
    pltpu.sync_copy(dacc.at[pl.ds(ss * TPW, TPW)], zbuf)
    pltpu.sync_copy(zbuf, out.at[pl.ds(cc * NPAD + ss * TPW, TPW)])


ROWS_PER_T = NROWS_E // NS
SBC = 4
NSUP = ROWS_PER_T // SBC


@functools.partial(
    pl.kernel,
    mesh=_mesh,
    out_type=jax.ShapeDtypeStruct((NC, NPAD, HALF), jnp.float32),
    scratch_types=[
        pltpu.VMEM((SBC, K), jnp.int32),
        pltpu.VMEM((SBC, K), jnp.int32),
        pltpu.VMEM((SBC, K), jnp.int32),
        pltpu.VMEM((SBC, K), jnp.int32),
        pltpu.VMEM((K, HALF), jnp.float32),
        pltpu.VMEM((K, HALF), jnp.float32),
        pltpu.VMEM((K, HALF), jnp.float32),
        pltpu.VMEM((K, HALF), jnp.float32),
        pltpu.VMEM((TPW // 16, HALF), jnp.float32),
        pltpu.VMEM_SHARED((NPAD, HALF), jnp.float32),
        pltpu.SemaphoreType.DMA,
        pltpu.SemaphoreType.DMA,
        pltpu.SemaphoreType.DMA,
        pltpu.SemaphoreType.DMA,
        pltpu.SemaphoreType.DMA,
        pltpu.SemaphoreType.DMA,
        pltpu.SemaphoreType.DMA,
        pltpu.SemaphoreType.DMA,
        pltpu.SemaphoreType.DMA,
        pltpu.SemaphoreType.DMA,
    ],
    compiler_params=pltpu.CompilerParams(use_tc_tiling_on_sc=False),
)
def _edge_kernel(lo, hi, src2d, dst2d, out, s0v, d0v, s1v, d1v,
                 rows0, rows1, rows2, rows3, zbuf, acc,
                 sem_i0, sem_i1, sem_g0, sem_g1, sem_g2, sem_g3,
                 sem_s0, sem_s1, sem_s2, sem_s3):
    cc = lax.axis_index("c")
    ss = lax.axis_index("s")
    z16 = jnp.zeros((16,), jnp.float32)
    rows = (rows0, rows1, rows2, rows3)
    sem_g = (sem_g0, sem_g1, sem_g2, sem_g3)
    sem_s = (sem_s0, sem_s1, sem_s2, sem_s3)
    sem_i = (sem_i0, sem_i1)

    def fire_idx(sup, ibs, ibd, sem, pred=None):
        def do():
            rr = ss * ROWS_PER_T + sup * SBC
            pltpu.async_copy(src2d.at[pl.ds(rr, SBC)], ibs, sem)
            pltpu.async_copy(dst2d.at[pl.ds(rr, SBC)], ibd, sem)
        if pred is None:
            do()
        else:
            pl.when(pred)(do)

    def wait_idx(ibs, ibd, sem, pred=None):
        def do():
            pltpu.make_async_copy(src2d.at[pl.ds(0, SBC)], ibs, sem).wait()
            pltpu.make_async_copy(dst2d.at[pl.ds(0, SBC)], ibd, sem).wait()
        if pred is None:
            do()
        else:
            pl.when(pred)(do)

    fire_idx(0, s0v, d0v, sem_i[0])

    zrows = TPW // 16

    def initz(r, _):
        zbuf[r, pl.ds(0, 16)] = z16
        zbuf[r, pl.ds(16, 16)] = z16
        return 0

    lax.fori_loop(0, zrows, initz, 0)
    for j in range(16):
        pltpu.sync_copy(zbuf, acc.at[pl.ds(ss * TPW + j * zrows, zrows)])

    wait_idx(s0v, d0v, sem_i[0])
    fire_idx(1, s1v, d1v, sem_i[1])
    plsc.subcore_barrier()

    def run(tbl):
        def fire_g(ib, j, b, pred=None):
            def do():
                pltpu.async_copy(tbl.at[ib.at[j]], rows[b], sem_g[b])
            if pred is None:
                do()
            else:
                pl.when(pred)(do)

        def drain_g(b):
            pltpu.make_async_copy(tbl.at[s0v.at[0]], rows[b], sem_g[b]).wait()

        def fire_s(ibd, j, b):
            pltpu.async_copy(rows[b], acc.at[ibd.at[j]], sem_s[b], add=True)

        def drain_s(b, pred=None):
            def do():
                pltpu.make_async_copy(rows[b], acc.at[d0v.at[0]],
                                      sem_s[b]).wait()
            if pred is None:
                do()
            else:
                pl.when(pred)(do)

        fire_g(s0v, 0, 0)
        fire_g(s0v, 1, 1)

        def body(t, _):
            not_first = t > 0
            not_last = t < NSUP // 2 - 1
            drain_s(2, pred=not_first)
            fire_g(s0v, 2, 2)
            drain_g(0)
            fire_s(d0v, 0, 0)
            drain_s(3, pred=not_first)
            fire_g(s0v, 3, 3)
            drain_g(1)
            fire_s(d0v, 1, 1)
            wait_idx(s1v, d1v, sem_i[1])
            drain_s(0)
            fire_g(s1v, 0, 0)
            drain_g(2)
            fire_s(d0v, 2, 2)
            drain_s(1)
            fire_g(s1v, 1, 1)
            drain_g(3)
            fire_s(d0v, 3, 3)
            fire_idx(2 * t + 2, s0v, d0v, sem_i[0], pred=not_last)
            drain_s(2)
            fire_g(s1v, 2, 2)
            drain_g(0)
            fire_s(d1v, 0, 0)
            drain_s(3)
            fire_g(s1v, 3, 3)
            drain_g(1)
            fire_s(d1v, 1, 1)
            wait_idx(s0v, d0v, sem_i[0], pred=not_last)
            drain_s(0)
            fire_g(s0v, 0, 0, pred=not_last)
            drain_g(2)
            fire_s(d1v, 2, 2)
            drain_s(1)
            fire_g(s0v, 1, 1, pred=not_last)
            drain_g(3)
            fire_s(d1v, 3, 3)
            fire_idx(2 * t + 3, s1v, d1v, sem_i[1], pred=not_last)
            return 0

        lax.fori_loop(0, NSUP // 2, body, 0)
        drain_s(2)
        drain_s(3)

    @pl.when(cc == 0)
    def _():
        run(lo)

    @pl.when(cc == 1)
    def _():
        run(hi)

    plsc.subcore_barrier()
    for j in range(16):
        pltpu.sync_copy(acc.at[pl.ds(ss * TPW + j * zrows, zrows)], zbuf)
        pltpu.sync_copy(zbuf, out.at[cc, pl.ds(ss * TPW + j * zrows, zrows)])


def _dinv_from(dp):
    deg = 1.0 + dp[0, 0, 0] + dp[1, 0, 0]
    return lax.rsqrt(deg)


def _mm1_body(x_ref, w1_ref, xw_ref):
    xw_ref[...] = jnp.dot(x_ref[...], w1_ref[...],
                          preferred_element_type=jnp.float32)


def _scale1_body(xw_ref, dp_ref, lo_ref, hi_ref):
    dinv = _dinv_from(dp_ref[...])
    xws = xw_ref[...] * dinv[:, None]
    lo_ref[...] = xws[:, :HALF]
    hi_ref[...] = xws[:, HALF:]


def _dense2_body(acc_ref, xw_ref, dp_ref, b1_ref, w2_ref,
                 xw2_ref, lo_ref, hi_ref):
    dinv = _dinv_from(dp_ref[...])
    xw = xw_ref[...]
    agg = jnp.concatenate([acc_ref[0], acc_ref[1]], axis=1)
    h = dinv[:, None] * agg + (dinv * dinv)[:, None] * xw + b1_ref[...]
    h = jnp.maximum(h, 0.0)
    xw2 = jnp.dot(h, w2_ref[...], preferred_element_type=jnp.float32)
    xws2 = xw2 * dinv[:, None]
    xw2_ref[...] = xw2
    lo_ref[...] = xws2[:, :HALF]
    hi_ref[...] = xws2[:, HALF:]


def _head_body(acc_ref, xw_ref, dp_ref, b2_ref, batch_ref, wfc_ref, bfc_ref,
               out_ref, sums_ref, cnt_ref):
    i = pl.program_id(0)
    dinv = _dinv_from(dp_ref[...])
    xw = xw_ref[...]
    agg = jnp.concatenate([acc_ref[0], acc_ref[1]], axis=1)
    h = dinv[:, None] * agg + (dinv * dinv)[:, None] * xw + b2_ref[...]
    h = jnp.maximum(h, 0.0)

    b = batch_ref[0, 0]
    onehot = (b[:, None] == lax.broadcasted_iota(jnp.int32, (BLK, N_GRAPHS),
                                                 1)).astype(jnp.float32)

    @pl.when(i == 0)
    def _():
        sums_ref[...] = jnp.zeros_like(sums_ref)
        cnt_ref[...] = jnp.zeros_like(cnt_ref)

    sums_ref[...] += lax.dot_general(onehot, h, (((0,), (0,)), ((), ())),
                                     preferred_element_type=jnp.float32)
    cnt_ref[...] += jnp.sum(onehot, axis=0, keepdims=True)

    @pl.when(i == NBLK - 1)
    def _():
        g = sums_ref[...] / jnp.maximum(cnt_ref[...], 1.0).reshape(
            N_GRAPHS, 1)
        res = jnp.sum(g * wfc_ref[...], axis=1) + bfc_ref[0, 0]
        out_ref[...] = res.reshape(1, N_GRAPHS)


def _mm1(x, W1):
    return pl.pallas_call(
        _mm1_body,
        grid=(NBLK,),
        in_specs=[
            pl.BlockSpec((BLK, D_IN), lambda i: (i, 0)),
            pl.BlockSpec((D_IN, D_H), lambda i: (0, 0)),
        ],
        out_specs=pl.BlockSpec((BLK, D_H), lambda i: (i, 0)),
        out_shape=jax.ShapeDtypeStruct((N, D_H), jnp.float32),
    )(x, W1)


def _scale1(xw, deg4):
    return pl.pallas_call(
        _scale1_body,
        grid=(NBLK,),
        in_specs=[
            pl.BlockSpec((BLK, D_H), lambda i: (i, 0)),
            pl.BlockSpec((2, 1, 1, BLK), lambda i: (0, i, 0, 0)),
        ],
        out_specs=[
            pl.BlockSpec((BLK, HALF), lambda i: (i, 0)),
            pl.BlockSpec((BLK, HALF), lambda i: (i, 0)),
        ],
        out_shape=[
            jax.ShapeDtypeStruct((N, HALF), jnp.float32),
            jax.ShapeDtypeStruct((N, HALF), jnp.float32),
        ],
    )(xw, deg4)


def _dense2(acc, xw, deg4, b1, W2):
    return pl.pallas_call(
        _dense2_body,
        grid=(NBLK,),
        in_specs=[
            pl.BlockSpec((2, BLK, HALF), lambda i: (0, i, 0)),
            pl.BlockSpec((BLK, D_H), lambda i: (i, 0)),
            pl.BlockSpec((2, 1, 1, BLK), lambda i: (0, i, 0, 0)),
            pl.BlockSpec((1, D_H), lambda i: (0, 0)),
            pl.BlockSpec((D_H, D_H), lambda i: (0, 0)),
        ],
        out_specs=[
            pl.BlockSpec((BLK, D_H), lambda i: (i, 0)),
            pl.BlockSpec((BLK, HALF), lambda i: (i, 0)),
            pl.BlockSpec((BLK, HALF), lambda i: (i, 0)),
        ],
        out_shape=[
            jax.ShapeDtypeStruct((N, D_H), jnp.float32),
            jax.ShapeDtypeStruct((N, HALF), jnp.float32),
            jax.ShapeDtypeStruct((N, HALF), jnp.float32),
        ],
    )(acc, xw, deg4, b1, W2)


def _head(acc, xw, deg4, b2, batch3, wfc_row, bfc2):
    return pl.pallas_call(
        _head_body,
        grid=(NBLK,),
        in_specs=[
            pl.BlockSpec((2, BLK, HALF), lambda i: (0, i, 0)),
            pl.BlockSpec((BLK, D_H), lambda i: (i, 0)),
            pl.BlockSpec((2, 1, 1, BLK), lambda i: (0, i, 0, 0)),
            pl.BlockSpec((1, D_H), lambda i: (0, 0)),
            pl.BlockSpec((1, 1, BLK), lambda i: (i, 0, 0)),
            pl.BlockSpec((1, D_H), lambda i: (0, 0)),
            pl.BlockSpec((1, 1), lambda i: (0, 0)),
        ],
        out_specs=pl.BlockSpec((1, N_GRAPHS), lambda i: (0, 0)),
        out_shape=jax.ShapeDtypeStruct((1, N_GRAPHS), jnp.float32),
        scratch_shapes=[
            pltpu.VMEM((N_GRAPHS, D_H), jnp.float32),
            pltpu.VMEM((1, N_GRAPHS), jnp.float32),
        ],
    )(acc, xw, deg4, b2, batch3, wfc_row, bfc2)


def kernel(x, edge_index, batch, W1, b1, W2, b2, Wfc, bfc):
    src = edge_index[0]
    dst = edge_index[1]
    pad = E_PAD - E
    src_p = jnp.concatenate([src, jnp.zeros((pad,), jnp.int32)])
    dst_p = jnp.concatenate([dst, jnp.full((pad,), N, jnp.int32)])
    src2d = src_p.reshape(NROWS_E, K)
    dst2d = dst_p.reshape(NROWS_E, K)

    deg_p = _deg_kernel(dst2d).reshape(NC, NPAD)
    deg4 = deg_p[:, :N].reshape(2, NBLK, 1, BLK)

    batch3 = batch.reshape(NBLK, 1, BLK)
    b1r = b1.reshape(1, D_H)
    b2r = b2.reshape(1, D_H)
    wfc_row = Wfc.reshape(1, D_H)
    bfc2 = bfc.reshape(1, 1)

    xw1 = _mm1(x, W1)
    lo1, hi1 = _scale1(xw1, deg4)
    acc1 = _edge_kernel(lo1, hi1, src2d, dst2d)
    xw2, lo2, hi2 = _dense2(acc1, xw1, deg4, b1r, W2)
    acc2 = _edge_kernel(lo2, hi2, src2d, dst2d)
    out2d = _head(acc2, xw2, deg4, b2r, batch3, wfc_row, bfc2)
    return out2d.reshape(N_GRAPHS)
